# baseline (speedup 1.0000x reference)
import jax
import jax.numpy as jnp
from jax.experimental import pallas as pl

_PACK = 4


def _embed_kernel(x_ref, pos_ref, w_ref, b_ref, o_ref):
    x = x_ref[...]
    w = w_ref[...]
    acc = jax.lax.dot_general(
        x, w, (((2,), (0,)), ((), ())),
        preferred_element_type=jnp.float32)
    o_ref[...] = acc + pos_ref[...][None, :, :] + b_ref[...][None, :, :]


def kernel(x, pos_table, W, b):
    B, L, D = x.shape
    Lp = L // _PACK
    Dp = D * _PACK

    x3 = x.reshape(B, Lp, Dp)
    pos2 = pos_table.reshape(Lp, Dp)
    b2 = jnp.tile(b, _PACK).reshape(1, Dp)

    wd = jnp.zeros((Dp, Dp), dtype=W.dtype)
    for i in range(_PACK):
        wd = wd.at[i * D:(i + 1) * D, i * D:(i + 1) * D].set(W)

    BB = 64
    out = pl.pallas_call(
        _embed_kernel,
        grid=(B // BB,),
        in_specs=[
            pl.BlockSpec((BB, Lp, Dp), lambda i: (i, 0, 0)),
            pl.BlockSpec((Lp, Dp), lambda i: (0, 0)),
            pl.BlockSpec((Dp, Dp), lambda i: (0, 0)),
            pl.BlockSpec((1, Dp), lambda i: (0, 0)),
        ],
        out_specs=pl.BlockSpec((BB, Lp, Dp), lambda i: (i, 0, 0)),
        out_shape=jax.ShapeDtypeStruct((B, Lp, Dp), x.dtype),
    )(x3, pos2, wd, b2)
    return out.reshape(B, L, D)
